# Initial kernel scaffold; baseline (speedup 1.0000x reference)
import jax
import jax.numpy as jnp
from jax.experimental import pallas as pl


def kernel(X, adj, W, a):
    raise NotImplementedError("write your pallas kernel here")


# speedup vs baseline: 1.6391x; 1.6391x over previous
import jax
import jax.numpy as jnp
from jax import lax
from jax.experimental import pallas as pl
from jax.experimental.pallas import tpu as pltpu

_N = 4096
_D = 128
_K = 4
_BR = 256
_SLOPE = 0.01


def _prep_body(x_ref, w_ref, a_ref, h_ref, s_ref):
    X = x_ref[:]
    for k in range(_K):
        Wk = w_ref[k]
        Hk = lax.dot_general(X, Wk, (((1,), (1,)), ((), ())),
                             preferred_element_type=jnp.float32)
        h_ref[k] = Hk
        a_nb = a_ref[k][:, :_D]
        s_row = lax.dot_general(a_nb, Hk, (((1,), (1,)), ((), ())),
                                preferred_element_type=jnp.float32)
        s_ref[pl.ds(k, 1), :] = s_row
    s_ref[pl.ds(_K, 8 - _K), :] = jnp.zeros((8 - _K, _N), jnp.float32)


def _attn_body(adj_ref, h_ref, s_ref, a_ref, out_ref):
    i = pl.program_id(0)
    mask = adj_ref[:] != 0.0
    neg_inf = jnp.float32(-jnp.inf)
    for k in range(_K):
        Hk = h_ref[k]
        Hrows = h_ref[k, pl.ds(i * _BR, _BR), :]
        a_self = a_ref[k][:, _D:]
        s_self = lax.dot_general(Hrows, a_self, (((1,), (1,)), ((), ())),
                                 preferred_element_type=jnp.float32)
        s_nb = s_ref[pl.ds(k, 1), :]
        scores = s_self + s_nb
        scores = jnp.where(scores >= 0.0, scores, _SLOPE * scores)
        masked = jnp.where(mask, scores, neg_inf)
        m = jnp.max(masked, axis=1, keepdims=True)
        p = jnp.where(mask, jnp.exp(scores - m), 0.0)
        denom = jnp.sum(p, axis=1, keepdims=True)
        alpha = p / denom
        out_ref[:, k * _D:(k + 1) * _D] = jnp.dot(
            alpha, Hk, preferred_element_type=jnp.float32)


@jax.jit
def kernel(X, adj, W, a):
    H, S = pl.pallas_call(
        _prep_body,
        grid=(1,),
        in_specs=[
            pl.BlockSpec((_N, _D), lambda i: (0, 0)),
            pl.BlockSpec((_K, _D, _D), lambda i: (0, 0, 0)),
            pl.BlockSpec((_K, 1, 2 * _D), lambda i: (0, 0, 0)),
        ],
        out_specs=[
            pl.BlockSpec((_K, _N, _D), lambda i: (0, 0, 0)),
            pl.BlockSpec((8, _N), lambda i: (0, 0)),
        ],
        out_shape=[
            jax.ShapeDtypeStruct((_K, _N, _D), jnp.float32),
            jax.ShapeDtypeStruct((8, _N), jnp.float32),
        ],
    )(X, W, a)

    out = pl.pallas_call(
        _attn_body,
        grid=(_N // _BR,),
        in_specs=[
            pl.BlockSpec((_BR, _N), lambda i: (i, 0)),
            pl.BlockSpec((_K, _N, _D), lambda i: (0, 0, 0)),
            pl.BlockSpec((8, _N), lambda i: (0, 0)),
            pl.BlockSpec((_K, 1, 2 * _D), lambda i: (0, 0, 0)),
        ],
        out_specs=pl.BlockSpec((_BR, _K * _D), lambda i: (i, 0)),
        out_shape=jax.ShapeDtypeStruct((_N, _K * _D), jnp.float32),
        compiler_params=pltpu.CompilerParams(
            dimension_semantics=("arbitrary",)),
    )(adj, H, S, a)
    return out


# speedup vs baseline: 1.9945x; 1.2168x over previous
import jax
import jax.numpy as jnp
from jax import lax
from jax.experimental import pallas as pl
from jax.experimental.pallas import tpu as pltpu

_N = 4096
_D = 128
_K = 4
_BR = 256
_SLOPE = 0.01


def _prep_body(x_ref, w_ref, a_ref, h_ref, s_ref):
    X = x_ref[:]
    for k in range(_K):
        Wk = w_ref[k]
        Hk = lax.dot_general(X, Wk, (((1,), (1,)), ((), ())),
                             preferred_element_type=jnp.float32)
        h_ref[k] = Hk
        a_nb = a_ref[k][:, :_D]
        s_row = lax.dot_general(a_nb, Hk, (((1,), (1,)), ((), ())),
                                preferred_element_type=jnp.float32)
        s_ref[pl.ds(k, 1), :] = s_row
    s_ref[pl.ds(_K, 8 - _K), :] = jnp.zeros((8 - _K, _N), jnp.float32)


def _attn_body(adj_ref, h_ref, s_ref, a_ref, out_ref):
    i = pl.program_id(0)
    mask = adj_ref[:] != 0.0
    neg_inf = jnp.float32(-jnp.inf)
    for k in range(_K):
        Hk = h_ref[k]
        Hrows = h_ref[k, pl.ds(i * _BR, _BR), :]
        a_self = a_ref[k][:, _D:]
        s_self = lax.dot_general(Hrows, a_self, (((1,), (1,)), ((), ())),
                                 preferred_element_type=jnp.float32)
        s_nb = s_ref[pl.ds(k, 1), :]
        scores = s_self + s_nb
        scores = jnp.maximum(scores, _SLOPE * scores)
        masked = jnp.where(mask, scores, neg_inf)
        m = jnp.max(masked, axis=1, keepdims=True)
        p = jnp.exp(masked - m)
        denom = jnp.sum(p, axis=1, keepdims=True)
        o = jnp.dot(p, Hk, preferred_element_type=jnp.float32)
        out_ref[:, k * _D:(k + 1) * _D] = o * (1.0 / denom)


@jax.jit
def kernel(X, adj, W, a):
    H, S = pl.pallas_call(
        _prep_body,
        grid=(1,),
        in_specs=[
            pl.BlockSpec((_N, _D), lambda i: (0, 0)),
            pl.BlockSpec((_K, _D, _D), lambda i: (0, 0, 0)),
            pl.BlockSpec((_K, 1, 2 * _D), lambda i: (0, 0, 0)),
        ],
        out_specs=[
            pl.BlockSpec((_K, _N, _D), lambda i: (0, 0, 0)),
            pl.BlockSpec((8, _N), lambda i: (0, 0)),
        ],
        out_shape=[
            jax.ShapeDtypeStruct((_K, _N, _D), jnp.float32),
            jax.ShapeDtypeStruct((8, _N), jnp.float32),
        ],
    )(X, W, a)

    out = pl.pallas_call(
        _attn_body,
        grid=(_N // _BR,),
        in_specs=[
            pl.BlockSpec((_BR, _N), lambda i: (i, 0)),
            pl.BlockSpec((_K, _N, _D), lambda i: (0, 0, 0)),
            pl.BlockSpec((8, _N), lambda i: (0, 0)),
            pl.BlockSpec((_K, 1, 2 * _D), lambda i: (0, 0, 0)),
        ],
        out_specs=pl.BlockSpec((_BR, _K * _D), lambda i: (i, 0)),
        out_shape=jax.ShapeDtypeStruct((_N, _K * _D), jnp.float32),
        compiler_params=pltpu.CompilerParams(
            dimension_semantics=("parallel",)),
    )(adj, H, S, a)
    return out


# speedup vs baseline: 2.2374x; 1.1218x over previous
import jax
import jax.numpy as jnp
from jax import lax
from jax.experimental import pallas as pl
from jax.experimental.pallas import tpu as pltpu

_N = 4096
_D = 128
_DE = 256
_K = 4
_BR = 256
_SLOPE = 0.01
_LOG2E = 1.4426950408889634


def _prep_body(x_ref, w_ref, a_ref, h_ref, s_ref):
    X = x_ref[:]
    lane = lax.broadcasted_iota(jnp.int32, (_N, _D), 1)
    denom_tile = jnp.where(lane == 0, 1.0, 0.0).astype(jnp.float32)
    for k in range(_K):
        Wk = w_ref[k]
        Hk = lax.dot_general(X, Wk, (((1,), (1,)), ((), ())),
                             preferred_element_type=jnp.float32)
        h_ref[k] = jnp.concatenate([Hk, denom_tile], axis=1)
        a_nb = a_ref[k][:, :_D] * _LOG2E
        s_row = lax.dot_general(a_nb, Hk, (((1,), (1,)), ((), ())),
                                preferred_element_type=jnp.float32)
        s_ref[pl.ds(k, 1), :] = s_row
    s_ref[pl.ds(_K, 8 - _K), :] = jnp.zeros((8 - _K, _N), jnp.float32)


def _attn_body(adj_ref, h_ref, s_ref, a_ref, out_ref):
    i = pl.program_id(0)
    off = (adj_ref[:] - 1.0) * jnp.float32(1e38)
    for k in range(_K):
        He = h_ref[k]
        Hrows = h_ref[k, pl.ds(i * _BR, _BR), :_D]
        a_self = a_ref[k][:, _D:] * _LOG2E
        s_self = lax.dot_general(Hrows, a_self, (((1,), (1,)), ((), ())),
                                 preferred_element_type=jnp.float32)
        s_nb = s_ref[pl.ds(k, 1), :]
        scores = s_self + s_nb
        scores = jnp.maximum(scores, _SLOPE * scores)
        masked = scores + off
        m = jnp.max(masked, axis=1, keepdims=True)
        p = jnp.exp2(masked - m)
        o_ext = jnp.dot(p, He, preferred_element_type=jnp.float32)
        o = o_ext[:, :_D]
        denom = o_ext[:, _D:_D + 1]
        out_ref[:, k * _D:(k + 1) * _D] = o * (1.0 / denom)


@jax.jit
def kernel(X, adj, W, a):
    H, S = pl.pallas_call(
        _prep_body,
        grid=(1,),
        in_specs=[
            pl.BlockSpec((_N, _D), lambda i: (0, 0)),
            pl.BlockSpec((_K, _D, _D), lambda i: (0, 0, 0)),
            pl.BlockSpec((_K, 1, 2 * _D), lambda i: (0, 0, 0)),
        ],
        out_specs=[
            pl.BlockSpec((_K, _N, _DE), lambda i: (0, 0, 0)),
            pl.BlockSpec((8, _N), lambda i: (0, 0)),
        ],
        out_shape=[
            jax.ShapeDtypeStruct((_K, _N, _DE), jnp.float32),
            jax.ShapeDtypeStruct((8, _N), jnp.float32),
        ],
    )(X, W, a)

    out = pl.pallas_call(
        _attn_body,
        grid=(_N // _BR,),
        in_specs=[
            pl.BlockSpec((_BR, _N), lambda i: (i, 0)),
            pl.BlockSpec((_K, _N, _DE), lambda i: (0, 0, 0)),
            pl.BlockSpec((8, _N), lambda i: (0, 0)),
            pl.BlockSpec((_K, 1, 2 * _D), lambda i: (0, 0, 0)),
        ],
        out_specs=pl.BlockSpec((_BR, _K * _D), lambda i: (i, 0)),
        out_shape=jax.ShapeDtypeStruct((_N, _K * _D), jnp.float32),
        compiler_params=pltpu.CompilerParams(
            dimension_semantics=("parallel",)),
    )(adj, H, S, a)
    return out


# speedup vs baseline: 2.5139x; 1.1236x over previous
import jax
import jax.numpy as jnp
from jax import lax
from jax.experimental import pallas as pl
from jax.experimental.pallas import tpu as pltpu

_N = 4096
_D = 128
_DE = 256
_K = 4
_BR = 256
_NB = _N // _BR
_SLOPE = 0.01
_LOG2E = 1.4426950408889634


def _gat_body(x_ref, w_ref, a_ref, adj_ref, out_ref, h_ref, s_ref):
    step = pl.program_id(0)

    @pl.when(step == 0)
    def _prep():
        X = x_ref[:]
        lane = lax.broadcasted_iota(jnp.int32, (_N, _D), 1)
        denom_tile = jnp.where(lane == 0, 1.0, 0.0).astype(jnp.float32)
        for k in range(_K):
            Wk = w_ref[k]
            Hk = lax.dot_general(X, Wk, (((1,), (1,)), ((), ())),
                                 preferred_element_type=jnp.float32)
            h_ref[k] = jnp.concatenate([Hk, denom_tile], axis=1)
            a_nb = a_ref[k][:, :_D] * _LOG2E
            a_self = a_ref[k][:, _D:] * _LOG2E
            s_ref[pl.ds(k, 1), :] = lax.dot_general(
                a_nb, Hk, (((1,), (1,)), ((), ())),
                preferred_element_type=jnp.float32)
            s_ref[pl.ds(_K + k, 1), :] = lax.dot_general(
                a_self, Hk, (((1,), (1,)), ((), ())),
                preferred_element_type=jnp.float32)

    @pl.when(step > 0)
    def _attn():
        i = step - 1
        mask = adj_ref[:] != 0.0
        neg_inf = jnp.float32(-jnp.inf)
        for k in range(_K):
            He = h_ref[k]
            s_nb = s_ref[pl.ds(k, 1), :]
            s_self_row = s_ref[pl.ds(_K + k, 1), pl.ds(i * _BR, _BR)]
            s_self = lax.transpose(s_self_row, (1, 0))
            scores = s_self + s_nb
            scores = jnp.maximum(scores, _SLOPE * scores)
            masked = jnp.where(mask, scores, neg_inf)
            m = jnp.max(masked, axis=1, keepdims=True)
            p = jnp.exp2(masked - m)
            o_ext = jnp.dot(p, He, preferred_element_type=jnp.float32)
            o = o_ext[:, :_D]
            denom = o_ext[:, _D:_D + 1]
            out_ref[:, k * _D:(k + 1) * _D] = o * (1.0 / denom)


@jax.jit
def kernel(X, adj, W, a):
    out = pl.pallas_call(
        _gat_body,
        grid=(1 + _NB,),
        in_specs=[
            pl.BlockSpec((_N, _D), lambda i: (0, 0)),
            pl.BlockSpec((_K, _D, _D), lambda i: (0, 0, 0)),
            pl.BlockSpec((_K, 1, 2 * _D), lambda i: (0, 0, 0)),
            pl.BlockSpec((_BR, _N), lambda i: (lax.max(i - 1, 0), 0)),
        ],
        out_specs=pl.BlockSpec((_BR, _K * _D), lambda i: (lax.max(i - 1, 0), 0)),
        out_shape=jax.ShapeDtypeStruct((_N, _K * _D), jnp.float32),
        scratch_shapes=[
            pltpu.VMEM((_K, _N, _DE), jnp.float32),
            pltpu.VMEM((8, _N), jnp.float32),
        ],
        compiler_params=pltpu.CompilerParams(
            dimension_semantics=("arbitrary",)),
    )(X, W, a, adj)
    return out
